# baseline (speedup 1.0000x reference)
import functools

import jax
import jax.numpy as jnp
from jax import lax
from jax.experimental import pallas as pl
from jax.experimental.pallas import tpu as pltpu
from jax.experimental.pallas import tpu_sc as plsc

H, W = 1080, 1920
C = 3
WC = W * C
L = 16
WPIX = W // L
R = 10
HALO = R + 20
NCHUNK = H // R
NC, NS = 2, 16
NW = NC * NS
CHUNKS_PER_TILE = (NCHUNK + NW - 1) // NW
TOL = 0.05

LOG2E = 1.4426950408889634
CE = (0.9999999967722976, 0.6931469985790312, 0.2402240280111151,
      0.05549016184393533, 0.009579040584883272, 0.0012752581358904615,
      0.00010935627255174577)
CL = (5.629329963841023e-07, 0.9999574661580921, -0.4992063824052593,
      0.3269723524219558, -0.22283471747775338, 0.13076335879445652,
      -0.052623955162732786, 0.01011890169509671)


def _sc_body(gs_hbm, gt_hbm, pr_hbm, out_hbm, gtb, prb, gxb, gyb, accb):
    cid = lax.axis_index("c")
    sid = lax.axis_index("s")
    wid = sid * NC + cid

    zero = jnp.zeros((L,), jnp.float32)
    accb[pl.ds(0, L)] = zero
    accb[pl.ds(L, L)] = zero
    accb[pl.ds(2 * L, L)] = zero

    lane = lax.iota(jnp.int32, L)
    expk = [lax.div(k * L + lane, 3) for k in range(3)]

    def do_chunk(chunk):
        base = chunk * R
        start = jnp.clip(base - 10, 0, H - HALO)
        startw = start * W
        pltpu.sync_copy(gt_hbm.at[pl.ds(startw, HALO * W)], gtb)
        pltpu.sync_copy(pr_hbm.at[pl.ds(startw, HALO * W)], prb)

        def row_body(r, carry):
            row = base + r
            pltpu.sync_copy(gs_hbm.at[pl.ds(row * WC, WC)], gxb)
            pltpu.sync_copy(gs_hbm.at[pl.ds((H + row) * WC, WC)], gyb)
            rowoff = (row - start) * W

            def grp(j, acc):
                a0, a1, a2 = acc
                srcbase = rowoff + j * L
                for k in range(3):
                    col = j * 3 * L + k * L
                    gxv = gxb[pl.ds(col, L)]
                    gyv = gyb[pl.ds(col, L)]
                    tidx = gyv * W + gxv - startw
                    tg = plsc.load_gather(gtb, [tidx])
                    tp = plsc.load_gather(prb, [tidx])
                    sidx = srcbase + expk[k]
                    sg = plsc.load_gather(gtb, [sidx])
                    sp = plsc.load_gather(prb, [sidx])

                    tgp = tg + 1e-8
                    pos = sg >= (1.0 + TOL) * tgp
                    neg = (1.0 + TOL) * sg <= tgp
                    nz = pos | neg
                    diff = sp - tp
                    ad = jnp.abs(diff)
                    x = jnp.maximum(ad * (-LOG2E), -126.0)
                    ki = x.astype(jnp.int32)
                    fr = x - ki.astype(jnp.float32)
                    p2 = jnp.float32(CE[6])
                    for c in (CE[5], CE[4], CE[3], CE[2], CE[1], CE[0]):
                        p2 = p2 * fr + c
                    scale = plsc.bitcast((ki + 127) << 23, jnp.float32)
                    e = p2 * scale
                    l = jnp.float32(CL[7])
                    for c in (CL[6], CL[5], CL[4], CL[3], CL[2], CL[1],
                              CL[0]):
                        l = l * e + c
                    t = jnp.where(pos, -diff, diff)
                    soft = jnp.maximum(t, 0.0) + l
                    a0 = a0 + jnp.where(nz, 1.0, 0.0)
                    a1 = a1 + jnp.where(nz, soft, 0.0)
                    a2 = a2 + jnp.where(nz, 0.0, diff * diff)
                return a0, a1, a2

            return plsc.parallel_loop(0, WPIX, unroll=2, carry=carry)(grp)

        a0, a1, a2 = lax.fori_loop(0, R, row_body, (zero, zero, zero))
        accb[pl.ds(0, L)] = accb[pl.ds(0, L)] + a0
        accb[pl.ds(L, L)] = accb[pl.ds(L, L)] + a1
        accb[pl.ds(2 * L, L)] = accb[pl.ds(2 * L, L)] + a2

    def chunk_body(ci, _):
        chunk = wid + ci * NW

        @pl.when(chunk < NCHUNK)
        def _():
            do_chunk(chunk)

        return 0

    lax.fori_loop(0, CHUNKS_PER_TILE, chunk_body, 0)
    pltpu.sync_copy(accb, out_hbm.at[pl.ds(wid * 3 * L, 3 * L)])


@functools.partial(
    pl.kernel,
    out_type=jax.ShapeDtypeStruct((NW * 3 * L,), jnp.float32),
    mesh=plsc.VectorSubcoreMesh(core_axis_name="c", subcore_axis_name="s"),
    compiler_params=pltpu.CompilerParams(needs_layout_passes=False),
    scratch_types=[
        pltpu.VMEM((HALO * W,), jnp.float32),
        pltpu.VMEM((HALO * W,), jnp.float32),
        pltpu.VMEM((WC,), jnp.int32),
        pltpu.VMEM((WC,), jnp.int32),
        pltpu.VMEM((3 * L,), jnp.float32),
    ],
)
def _depth_loss_partials(gs_hbm, gt_hbm, pr_hbm, out_hbm, gtb, prb, gxb, gyb,
                         accb):
    _sc_body(gs_hbm, gt_hbm, pr_hbm, out_hbm, gtb, prb, gxb, gyb, accb)


def kernel(pred_depth, gt_depth, grid, grid_shift):
    gs = grid_shift.reshape(2 * H * WC)
    parts = _depth_loss_partials(gs, gt_depth.reshape(H * W),
                                 pred_depth.reshape(H * W))
    parts = parts.reshape(NW, 3, L)
    n_nz = jnp.sum(parts[:, 0])
    s_soft = jnp.sum(parts[:, 1])
    s_sq = jnp.sum(parts[:, 2])
    total = jnp.float32(H * WC)
    depth_loss = s_soft / jnp.maximum(n_nz, 1.0)
    depth_loss_sim = s_sq / jnp.maximum(total - n_nz, 1.0)
    return depth_loss + depth_loss_sim


# speedup vs baseline: 1.9525x; 1.9525x over previous
import functools

import jax
import jax.numpy as jnp
from jax import lax
from jax.experimental import pallas as pl
from jax.experimental.pallas import tpu as pltpu
from jax.experimental.pallas import tpu_sc as plsc

H, W = 1080, 1920
C = 3
WC = W * C
HWC = WC // 2
L = 16
JGRP = HWC // (3 * L)
RING = 32
NC, NS = 2, 16
NW = NC * NS
TOL = 0.05
YSH = 11

LOG2E = 1.4426950408889634
CE = (0.9999961199303905, 0.6930292690219008, 0.23938500062629817,
      0.05318647048254017, 0.006838262187515184)
CL = (2.2132784000594707e-05, 0.9990102089269637, -0.4891557820114497,
      0.28330238362040977, -0.1301179302884552, 0.030102247599666062)


def _softplus_terms(tg, tp, sg, sp):
    tgp = tg + 1e-8
    pos = sg >= (1.0 + TOL) * tgp
    neg = (1.0 + TOL) * sg <= tgp
    nz = pos | neg
    diff = sp - tp
    ad = jnp.abs(diff)
    x = jnp.maximum(ad * (-LOG2E), -126.0)
    ki = x.astype(jnp.int32)
    fr = x - ki.astype(jnp.float32)
    p2 = jnp.float32(CE[4])
    for c in (CE[3], CE[2], CE[1], CE[0]):
        p2 = p2 * fr + c
    scale = plsc.bitcast((ki + 127) << 23, jnp.float32)
    e = p2 * scale
    l = jnp.float32(CL[5])
    for c in (CL[4], CL[3], CL[2], CL[1], CL[0]):
        l = l * e + c
    t = jnp.where(pos, -diff, diff)
    soft = jnp.maximum(t, 0.0) + l
    m = jnp.where(nz, 1.0, 0.0)
    s = jnp.where(nz, soft, 0.0)
    q = jnp.where(nz, 0.0, diff * diff)
    return m, s, q


def _sc_body(pk_hbm, gt_hbm, pr_hbm, out_hbm, gtb, prb, pka, pkb, accb,
             sema, semb):
    cid = lax.axis_index("c")
    sid = lax.axis_index("s")
    wid = sid * NC + cid

    lo = (wid * H) // NW
    hi = ((wid + 1) * H) // NW

    zero = jnp.zeros((L,), jnp.float32)
    lane = lax.iota(jnp.int32, L)
    expk = [lax.div(k * L + lane, 3) for k in range(3)]

    def load_group(g):
        g = pl.multiple_of(g, 8)
        slot = pl.multiple_of(g & (RING - 1), 8)
        pltpu.sync_copy(gt_hbm.at[pl.ds(g, 8)], gtb.at[pl.ds(slot, 8)])
        pltpu.sync_copy(pr_hbm.at[pl.ds(g, 8)], prb.at[pl.ds(slot, 8)])

    g0 = jnp.maximum(lo - 10, 0) // 8
    g1 = (lo + 10) // 8

    def pre_body(gi, _):
        load_group(gi * 8)
        return 0

    lax.fori_loop(g0, g1 + 1, pre_body, 0)

    def process_half(r, buf, half, carry):
        slotv = jnp.full((L,), r & (RING - 1), jnp.int32)
        pbase = half * (HWC // 3)

        def grp(j, acc):
            a0, a1, a2 = acc
            srcpix = pbase + j * L
            for k in range(3):
                col = j * 3 * L + k * L
                pkv = buf[pl.ds(col, L)]
                gyv = pkv >> YSH
                gxv = pkv & (2 ** YSH - 1)
                lslot = gyv & (RING - 1)
                tg = plsc.load_gather(gtb, [lslot, gxv])
                tp = plsc.load_gather(prb, [lslot, gxv])
                spix = srcpix + expk[k]
                sg = plsc.load_gather(gtb, [slotv, spix])
                sp = plsc.load_gather(prb, [slotv, spix])
                m, s, q = _softplus_terms(tg, tp, sg, sp)
                a0 = a0 + m
                a1 = a1 + s
                a2 = a2 + q
            return a0, a1, a2

        return plsc.parallel_loop(0, JGRP, unroll=2, carry=carry)(grp)

    pltpu.async_copy(pk_hbm.at[pl.ds(lo * WC, HWC)], pka, sema)

    def row_body(r, carry):
        @pl.when((((r + 10) & 7) == 0) & (r + 10 <= H - 8))
        def _():
            load_group(r + 10)

        pltpu.async_copy(pk_hbm.at[pl.ds(r * WC + HWC, HWC)], pkb, semb)
        pltpu.make_async_copy(pk_hbm.at[pl.ds(r * WC, HWC)], pka,
                              sema).wait()
        carry = process_half(r, pka, 0, carry)

        @pl.when(r + 1 < hi)
        def _():
            pltpu.async_copy(pk_hbm.at[pl.ds((r + 1) * WC, HWC)], pka, sema)

        pltpu.make_async_copy(pk_hbm.at[pl.ds(r * WC + HWC, HWC)], pkb,
                              semb).wait()
        carry = process_half(r, pkb, 1, carry)
        return carry

    a0, a1, a2 = lax.fori_loop(lo, hi, row_body, (zero, zero, zero))
    accb[pl.ds(0, L)] = a0
    accb[pl.ds(L, L)] = a1
    accb[pl.ds(2 * L, L)] = a2
    pltpu.sync_copy(accb, out_hbm.at[pl.ds(wid * 3 * L, 3 * L)])


@functools.partial(
    pl.kernel,
    out_type=jax.ShapeDtypeStruct((NW * 3 * L,), jnp.float32),
    mesh=plsc.VectorSubcoreMesh(core_axis_name="c", subcore_axis_name="s"),
    compiler_params=pltpu.CompilerParams(needs_layout_passes=False),
    scratch_types=[
        pltpu.VMEM((RING, W), jnp.float32),
        pltpu.VMEM((RING, W), jnp.float32),
        pltpu.VMEM((HWC,), jnp.int32),
        pltpu.VMEM((HWC,), jnp.int32),
        pltpu.VMEM((3 * L,), jnp.float32),
        pltpu.SemaphoreType.DMA,
        pltpu.SemaphoreType.DMA,
    ],
)
def _depth_loss_partials(pk_hbm, gt_hbm, pr_hbm, out_hbm, gtb, prb, pka,
                         pkb, accb, sema, semb):
    _sc_body(pk_hbm, gt_hbm, pr_hbm, out_hbm, gtb, prb, pka, pkb, accb,
             sema, semb)


def kernel(pred_depth, gt_depth, grid, grid_shift):
    pk = ((grid_shift[1] << YSH) | grid_shift[0]).reshape(H * WC)
    parts = _depth_loss_partials(pk, gt_depth, pred_depth)
    parts = parts.reshape(NW, 3, L)
    n_nz = jnp.sum(parts[:, 0])
    s_soft = jnp.sum(parts[:, 1])
    s_sq = jnp.sum(parts[:, 2])
    total = jnp.float32(H * WC)
    depth_loss = s_soft / jnp.maximum(n_nz, 1.0)
    depth_loss_sim = s_sq / jnp.maximum(total - n_nz, 1.0)
    return depth_loss + depth_loss_sim


# speedup vs baseline: 49.3385x; 25.2694x over previous
import functools

import jax
import jax.numpy as jnp
from jax import lax
from jax.experimental import pallas as pl
from jax.experimental.pallas import tpu as pltpu
from jax.experimental.pallas import tpu_sc as plsc

H, W = 1080, 1920
C = 3
L = 16
RING = 32
SW = 640
CW = 896
CS = (0, 512, 1024)
SLOC = (0, 128, 256)
JG = SW // L
NC, NS = 2, 16
NW = NC * NS
NGRP = 6
TOL = 0.05
YSH = 11

LOG2E = 1.4426950408889634
CE = (0.9999961199303905, 0.6930292690219008, 0.23938500062629817,
      0.05318647048254017, 0.006838262187515184)
CL = (2.2132784000594707e-05, 0.9990102089269637, -0.4891557820114497,
      0.28330238362040977, -0.1301179302884552, 0.030102247599666062)


def _softplus_terms(tg, tp, sg, sp):
    tgp = tg + 1e-8
    pos = sg >= (1.0 + TOL) * tgp
    neg = (1.0 + TOL) * sg <= tgp
    nz = pos | neg
    diff = sp - tp
    ad = jnp.abs(diff)
    x = jnp.maximum(ad * (-LOG2E), -126.0)
    ki = x.astype(jnp.int32)
    fr = x - ki.astype(jnp.float32)
    p2 = jnp.float32(CE[4])
    for c in (CE[3], CE[2], CE[1], CE[0]):
        p2 = p2 * fr + c
    scale = plsc.bitcast((ki + 127) << 23, jnp.float32)
    e = p2 * scale
    l = jnp.float32(CL[5])
    for c in (CL[4], CL[3], CL[2], CL[1], CL[0]):
        l = l * e + c
    t = jnp.where(pos, -diff, diff)
    soft = jnp.maximum(t, 0.0) + l
    m = jnp.where(nz, 1.0, 0.0)
    s = jnp.where(nz, soft, 0.0)
    q = jnp.where(nz, 0.0, diff * diff)
    return m, s, q


def _sc_body(pk_hbm, gt_hbm, pr_hbm, out_hbm, gtb, prb, ixa, ixb, accb,
             sema, semb):
    cid = lax.axis_index("c")
    sid = lax.axis_index("s")
    wid = sid * NC + cid

    lo = (wid * H) // NW
    hi = ((wid + 1) * H) // NW
    glo = lo // 8

    zero = jnp.zeros((L,), jnp.float32)

    def strip_pass(s, acc):
        cs = CS[s]
        sloc = SLOC[s]

        def load_ring_group(g):
            g = pl.multiple_of(g, 8)
            slot = pl.multiple_of(g & (RING - 1), 8)
            pltpu.sync_copy(gt_hbm.at[pl.ds(g, 8), pl.ds(cs, CW)],
                            gtb.at[pl.ds(slot, 8)])
            pltpu.sync_copy(pr_hbm.at[pl.ds(g, 8), pl.ds(cs, CW)],
                            prb.at[pl.ds(slot, 8)])

        def idx_dma(gi, buf, sem):
            g8 = pl.multiple_of(jnp.minimum(gi * 8, H - 8), 8)
            return [
                pltpu.make_async_copy(
                    pk_hbm.at[c, pl.ds(g8, 8), pl.ds(s * SW, SW)],
                    buf.at[c], sem)
                for c in range(C)
            ]

        def idx_start(gi, buf, sem):
            for d in idx_dma(gi, buf, sem):
                d.start()

        def idx_wait(gi, buf, sem):
            for d in idx_dma(gi, buf, sem):
                d.wait()

        pg0 = jnp.maximum(lo - 10, 0) // 8
        pg1 = (lo + 10) // 8

        def pre_body(gi, _):
            load_ring_group(gi * 8)
            return 0

        lax.fori_loop(pg0, pg1 + 1, pre_body, 0)

        def process_group(gi, buf, acc):
            gbase = gi * 8
            rlo = jnp.maximum(lo, gbase)
            rhi = jnp.minimum(hi, gbase + 8)

            def row_body(r, carry):
                @pl.when((((r + 10) & 7) == 0) & (r + 10 <= H - 8))
                def _():
                    load_ring_group(r + 10)

                rloc = r - gbase
                slot_r = r & (RING - 1)

                def grp(j, a):
                    a0, a1, a2 = a
                    jcol = j * L
                    for c in range(C):
                        pkv = buf[c, rloc, pl.ds(jcol, L)]
                        gyv = pkv >> YSH
                        gxv = pkv & (2 ** YSH - 1)
                        lslot = gyv & (RING - 1)
                        lcol = gxv - cs
                        tg = plsc.load_gather(gtb, [lslot, lcol])
                        tp = plsc.load_gather(prb, [lslot, lcol])
                        sg = gtb[slot_r, pl.ds(sloc + jcol, L)]
                        sp = prb[slot_r, pl.ds(sloc + jcol, L)]
                        m, sf, q = _softplus_terms(tg, tp, sg, sp)
                        a0 = a0 + m
                        a1 = a1 + sf
                        a2 = a2 + q
                    return a0, a1, a2

                return plsc.parallel_loop(0, JG, unroll=2, carry=carry)(grp)

            return lax.fori_loop(rlo, rhi, row_body, acc)

        idx_start(glo, ixa, sema)
        for gp in range(NGRP // 2):
            ga = glo + 2 * gp
            gb = ga + 1
            idx_start(gb, ixb, semb)
            idx_wait(ga, ixa, sema)
            acc = process_group(ga, ixa, acc)
            if gp < NGRP // 2 - 1:
                idx_start(ga + 2, ixa, sema)
            idx_wait(gb, ixb, semb)
            acc = process_group(gb, ixb, acc)
        return acc

    acc = (zero, zero, zero)
    for s in range(3):
        acc = strip_pass(s, acc)

    accb[pl.ds(0, L)] = acc[0]
    accb[pl.ds(L, L)] = acc[1]
    accb[pl.ds(2 * L, L)] = acc[2]
    pltpu.sync_copy(accb, out_hbm.at[pl.ds(wid * 3 * L, 3 * L)])


@functools.partial(
    pl.kernel,
    out_type=jax.ShapeDtypeStruct((NW * 3 * L,), jnp.float32),
    mesh=plsc.VectorSubcoreMesh(core_axis_name="c", subcore_axis_name="s"),
    compiler_params=pltpu.CompilerParams(needs_layout_passes=False),
    scratch_types=[
        pltpu.VMEM((RING, CW), jnp.float32),
        pltpu.VMEM((RING, CW), jnp.float32),
        pltpu.VMEM((C, 8, SW), jnp.int32),
        pltpu.VMEM((C, 8, SW), jnp.int32),
        pltpu.VMEM((3 * L,), jnp.float32),
        pltpu.SemaphoreType.DMA,
        pltpu.SemaphoreType.DMA,
    ],
)
def _depth_loss_partials(pk_hbm, gt_hbm, pr_hbm, out_hbm, gtb, prb, ixa,
                         ixb, accb, sema, semb):
    _sc_body(pk_hbm, gt_hbm, pr_hbm, out_hbm, gtb, prb, ixa, ixb, accb,
             sema, semb)


def kernel(pred_depth, gt_depth, grid, grid_shift):
    pk = jnp.stack([(grid_shift[1, :, :, c] << YSH) | grid_shift[0, :, :, c]
                    for c in range(C)])
    parts = _depth_loss_partials(pk, gt_depth, pred_depth)
    parts = parts.reshape(NW, 3, L)
    n_nz = jnp.sum(parts[:, 0])
    s_soft = jnp.sum(parts[:, 1])
    s_sq = jnp.sum(parts[:, 2])
    total = jnp.float32(H * W * C)
    depth_loss = s_soft / jnp.maximum(n_nz, 1.0)
    depth_loss_sim = s_sq / jnp.maximum(total - n_nz, 1.0)
    return depth_loss + depth_loss_sim


# speedup vs baseline: 55.1726x; 1.1182x over previous
import functools

import jax
import jax.numpy as jnp
from jax import lax
from jax.experimental import pallas as pl
from jax.experimental.pallas import tpu as pltpu
from jax.experimental.pallas import tpu_sc as plsc

H, W = 1080, 1920
C = 3
L = 16
RING = 32
SW = 640
CW = 896
CS = (0, 512, 1024)
SLOC = (0, 128, 256)
JG = SW // L
NC, NS = 2, 16
NW = NC * NS
NGRP = 6
TOL = 0.05

LOG2E = 1.4426950408889634
CE = (0.9999961199303905, 0.6930292690219008, 0.23938500062629817,
      0.05318647048254017, 0.006838262187515184)
CL = (2.2132784000594707e-05, 0.9990102089269637, -0.4891557820114497,
      0.28330238362040977, -0.1301179302884552, 0.030102247599666062)


def _softplus_terms(tg, tp, sg, sp):
    tgp = tg + 1e-8
    pos = sg >= (1.0 + TOL) * tgp
    neg = (1.0 + TOL) * sg <= tgp
    nz = pos | neg
    diff = sp - tp
    ad = jnp.abs(diff)
    x = jnp.maximum(ad * (-LOG2E), -126.0)
    ki = x.astype(jnp.int32)
    fr = x - ki.astype(jnp.float32)
    p2 = jnp.float32(CE[4])
    for c in (CE[3], CE[2], CE[1], CE[0]):
        p2 = p2 * fr + c
    scale = plsc.bitcast((ki + 127) << 23, jnp.float32)
    e = p2 * scale
    l = jnp.float32(CL[5])
    for c in (CL[4], CL[3], CL[2], CL[1], CL[0]):
        l = l * e + c
    t = jnp.where(pos, -diff, diff)
    soft = jnp.maximum(t, 0.0) + l
    m = jnp.where(nz, 1.0, 0.0)
    s = jnp.where(nz, soft, 0.0)
    q = jnp.where(nz, 0.0, diff * diff)
    return m, s, q


def _sc_body(gs_hbm, gt_hbm, pr_hbm, out_hbm, gtb, prb, gxa, gya,
             gxb, gyb, accb, sema, semb):
    cid = lax.axis_index("c")
    sid = lax.axis_index("s")
    wid = sid * NC + cid

    lo = (wid * H) // NW
    hi = ((wid + 1) * H) // NW
    glo = lo // 8

    zero = jnp.zeros((L,), jnp.float32)

    def strip_pass(s, acc):
        cs = CS[s]
        sloc = SLOC[s]

        def load_ring_group(g):
            g = pl.multiple_of(g, 8)
            slot = pl.multiple_of(g & (RING - 1), 8)
            pltpu.sync_copy(gt_hbm.at[pl.ds(g, 8), pl.ds(cs, CW)],
                            gtb.at[pl.ds(slot, 8)])
            pltpu.sync_copy(pr_hbm.at[pl.ds(g, 8), pl.ds(cs, CW)],
                            prb.at[pl.ds(slot, 8)])

        def idx_dma(gi, bufx, bufy, sem):
            g8 = pl.multiple_of(jnp.minimum(gi * 8, H - 8), 8)
            ds = []
            for c in range(C):
                ds.append(pltpu.make_async_copy(
                    gs_hbm.at[0, c, pl.ds(g8, 8), pl.ds(s * SW, SW)],
                    bufx.at[c], sem))
                ds.append(pltpu.make_async_copy(
                    gs_hbm.at[1, c, pl.ds(g8, 8), pl.ds(s * SW, SW)],
                    bufy.at[c], sem))
            return ds

        def idx_start(gi, bufx, bufy, sem):
            for d in idx_dma(gi, bufx, bufy, sem):
                d.start()

        def idx_wait(gi, bufx, bufy, sem):
            for d in idx_dma(gi, bufx, bufy, sem):
                d.wait()

        pg0 = jnp.maximum(lo - 10, 0) // 8
        pg1 = (lo + 10) // 8

        def pre_body(gi, _):
            load_ring_group(gi * 8)
            return 0

        lax.fori_loop(pg0, pg1 + 1, pre_body, 0)

        def process_group(gi, bufx, bufy, acc):
            gbase = gi * 8
            rlo = jnp.maximum(lo, gbase)
            rhi = jnp.minimum(hi, gbase + 8)

            def row_body(r, carry):
                @pl.when((((r + 10) & 7) == 0) & (r + 10 <= H - 8))
                def _():
                    load_ring_group(r + 10)

                rloc = r - gbase
                slot_r = r & (RING - 1)

                def grp(j, a):
                    a0, a1, a2 = a
                    jcol = j * L
                    for c in range(C):
                        gxv = bufx[c, rloc, pl.ds(jcol, L)]
                        gyv = bufy[c, rloc, pl.ds(jcol, L)]
                        lslot = gyv & (RING - 1)
                        lcol = gxv - cs
                        tg = plsc.load_gather(gtb, [lslot, lcol])
                        tp = plsc.load_gather(prb, [lslot, lcol])
                        sg = gtb[slot_r, pl.ds(sloc + jcol, L)]
                        sp = prb[slot_r, pl.ds(sloc + jcol, L)]
                        m, sf, q = _softplus_terms(tg, tp, sg, sp)
                        a0 = a0 + m
                        a1 = a1 + sf
                        a2 = a2 + q
                    return a0, a1, a2

                return plsc.parallel_loop(0, JG, unroll=2, carry=carry)(grp)

            return lax.fori_loop(rlo, rhi, row_body, acc)

        idx_start(glo, gxa, gya, sema)
        for gp in range(NGRP // 2):
            ga = glo + 2 * gp
            gb = ga + 1
            idx_start(gb, gxb, gyb, semb)
            idx_wait(ga, gxa, gya, sema)
            acc = process_group(ga, gxa, gya, acc)
            if gp < NGRP // 2 - 1:
                idx_start(ga + 2, gxa, gya, sema)
            idx_wait(gb, gxb, gyb, semb)
            acc = process_group(gb, gxb, gyb, acc)
        return acc

    acc = (zero, zero, zero)
    for s in range(3):
        acc = strip_pass(s, acc)

    accb[pl.ds(0, L)] = acc[0]
    accb[pl.ds(L, L)] = acc[1]
    accb[pl.ds(2 * L, L)] = acc[2]
    pltpu.sync_copy(accb, out_hbm.at[pl.ds(wid * 3 * L, 3 * L)])


@functools.partial(
    pl.kernel,
    out_type=jax.ShapeDtypeStruct((NW * 3 * L,), jnp.float32),
    mesh=plsc.VectorSubcoreMesh(core_axis_name="c", subcore_axis_name="s"),
    compiler_params=pltpu.CompilerParams(needs_layout_passes=False),
    scratch_types=[
        pltpu.VMEM((RING, CW), jnp.float32),
        pltpu.VMEM((RING, CW), jnp.float32),
        pltpu.VMEM((C, 8, SW), jnp.int32),
        pltpu.VMEM((C, 8, SW), jnp.int32),
        pltpu.VMEM((C, 8, SW), jnp.int32),
        pltpu.VMEM((C, 8, SW), jnp.int32),
        pltpu.VMEM((3 * L,), jnp.float32),
        pltpu.SemaphoreType.DMA,
        pltpu.SemaphoreType.DMA,
    ],
)
def _depth_loss_partials(gs_hbm, gt_hbm, pr_hbm, out_hbm, gtb, prb,
                         gxa, gya, gxb, gyb, accb, sema, semb):
    _sc_body(gs_hbm, gt_hbm, pr_hbm, out_hbm, gtb, prb, gxa, gya,
             gxb, gyb, accb, sema, semb)


def kernel(pred_depth, gt_depth, grid, grid_shift):
    gs4 = jnp.transpose(grid_shift, (0, 3, 1, 2))
    parts = _depth_loss_partials(gs4, gt_depth, pred_depth)
    parts = parts.reshape(NW, 3, L)
    n_nz = jnp.sum(parts[:, 0])
    s_soft = jnp.sum(parts[:, 1])
    s_sq = jnp.sum(parts[:, 2])
    total = jnp.float32(H * W * C)
    depth_loss = s_soft / jnp.maximum(n_nz, 1.0)
    depth_loss_sim = s_sq / jnp.maximum(total - n_nz, 1.0)
    return depth_loss + depth_loss_sim


# speedup vs baseline: 63.1882x; 1.1453x over previous
import functools

import jax
import jax.numpy as jnp
from jax import lax
from jax.experimental import pallas as pl
from jax.experimental.pallas import tpu as pltpu
from jax.experimental.pallas import tpu_sc as plsc

H, W = 1080, 1920
C = 3
L = 16
RING = 32
SW = 640
CW = 896
CS = (0, 512, 1024)
SLOC = (0, 128, 256)
JG = SW // L
NC, NS = 2, 16
NW = NC * NS
NGRP = 6
TOL = 0.05

LOG2E = 1.4426950408889634
CE = (0.9999961199303905, 0.6930292690219008, 0.23938500062629817,
      0.05318647048254017, 0.006838262187515184)
CL = (2.2132784000594707e-05, 0.9990102089269637, -0.4891557820114497,
      0.28330238362040977, -0.1301179302884552, 0.030102247599666062)


def _softplus_terms(tg, tp, sg, sgx, sp):
    tgp = tg + 1e-8
    pos = sg >= (1.0 + TOL) * tgp
    neg = sgx <= tgp
    nz = pos | neg
    diff = sp - tp
    ad = jnp.abs(diff)
    x = jnp.maximum(ad * (-LOG2E), -126.0)
    ki = x.astype(jnp.int32)
    fr = x - ki.astype(jnp.float32)
    p2 = jnp.float32(CE[4])
    for c in (CE[3], CE[2], CE[1], CE[0]):
        p2 = p2 * fr + c
    scale = plsc.bitcast((ki + 127) << 23, jnp.float32)
    e = p2 * scale
    l = jnp.float32(CL[5])
    for c in (CL[4], CL[3], CL[2], CL[1], CL[0]):
        l = l * e + c
    t = jnp.where(pos, -diff, diff)
    soft = jnp.maximum(t, 0.0) + l
    m = jnp.where(nz, 1.0, 0.0)
    s = jnp.where(nz, soft, 0.0)
    q = jnp.where(nz, 0.0, diff * diff)
    return m, s, q


def _sc_body(gs_hbm, gt_hbm, pr_hbm, out_hbm, gtb, prb, gxa, gya,
             gxb, gyb, accb, sema, semb, semg):
    cid = lax.axis_index("c")
    sid = lax.axis_index("s")
    wid = sid * NC + cid

    lo = (wid * H) // NW
    hi = ((wid + 1) * H) // NW
    glo = lo // 8

    zero = jnp.zeros((L,), jnp.float32)

    def strip_pass(s, acc):
        cs = CS[s]
        sloc = SLOC[s]

        def ring_dma(g):
            g = pl.multiple_of(g, 8)
            slot = pl.multiple_of(g & (RING - 1), 8)
            return [
                pltpu.make_async_copy(gt_hbm.at[pl.ds(g, 8), pl.ds(cs, CW)],
                                      gtb.at[pl.ds(slot, 8)], semg),
                pltpu.make_async_copy(pr_hbm.at[pl.ds(g, 8), pl.ds(cs, CW)],
                                      prb.at[pl.ds(slot, 8)], semg),
            ]

        def load_ring_group(g):
            for d in ring_dma(g):
                d.start()
            for d in ring_dma(g):
                d.wait()

        def idx_dma(gi, bufx, bufy, sem):
            g8 = pl.multiple_of(jnp.minimum(gi * 8, H - 8), 8)
            ds = []
            for c in range(C):
                ds.append(pltpu.make_async_copy(
                    gs_hbm.at[0, c, pl.ds(g8, 8), pl.ds(s * SW, SW)],
                    bufx.at[c], sem))
                ds.append(pltpu.make_async_copy(
                    gs_hbm.at[1, c, pl.ds(g8, 8), pl.ds(s * SW, SW)],
                    bufy.at[c], sem))
            return ds

        def idx_start(gi, bufx, bufy, sem):
            for d in idx_dma(gi, bufx, bufy, sem):
                d.start()

        def idx_wait(gi, bufx, bufy, sem):
            for d in idx_dma(gi, bufx, bufy, sem):
                d.wait()

        pg0 = jnp.maximum(lo - 10, 0) // 8
        pmax = (lo + 13) // 8

        def pre_body(gi, _):
            load_ring_group(gi * 8)
            return 0

        lax.fori_loop(pg0, pmax + 1, pre_body, 0)

        def process_group(gi, bufx, bufy, acc):
            gbase = gi * 8
            rlo = jnp.maximum(lo, gbase)
            rhi = jnp.minimum(hi, gbase + 8)

            def row_body(r, carry):
                @pl.when((((r + 14) & 7) == 0) & ((r + 14) // 8 > pmax)
                         & (r + 14 <= H - 8) & (r + 4 < hi))
                def _():
                    for d in ring_dma(r + 14):
                        d.start()

                @pl.when((((r + 10) & 7) == 0) & ((r + 10) // 8 > pmax)
                         & (r + 10 <= H - 8))
                def _():
                    for d in ring_dma(r + 10):
                        d.wait()

                rloc = r - gbase
                slot_r = r & (RING - 1)

                def grp(j, a):
                    a0, a1, a2 = a
                    jcol = j * L
                    sg = gtb[slot_r, pl.ds(sloc + jcol, L)]
                    sp = prb[slot_r, pl.ds(sloc + jcol, L)]
                    sgx = (1.0 + TOL) * sg
                    for c in range(C):
                        gxv = bufx[c, rloc, pl.ds(jcol, L)]
                        gyv = bufy[c, rloc, pl.ds(jcol, L)]
                        lslot = gyv & (RING - 1)
                        lcol = gxv - cs
                        tg = plsc.load_gather(gtb, [lslot, lcol])
                        tp = plsc.load_gather(prb, [lslot, lcol])
                        m, sf, q = _softplus_terms(tg, tp, sg, sgx, sp)
                        a0 = a0 + m
                        a1 = a1 + sf
                        a2 = a2 + q
                    return a0, a1, a2

                return plsc.parallel_loop(0, JG, unroll=2, carry=carry)(grp)

            return lax.fori_loop(rlo, rhi, row_body, acc)

        idx_start(glo, gxa, gya, sema)
        for gp in range(NGRP // 2):
            ga = glo + 2 * gp
            gb = ga + 1
            idx_start(gb, gxb, gyb, semb)
            idx_wait(ga, gxa, gya, sema)
            acc = process_group(ga, gxa, gya, acc)
            if gp < NGRP // 2 - 1:
                idx_start(ga + 2, gxa, gya, sema)
            idx_wait(gb, gxb, gyb, semb)
            acc = process_group(gb, gxb, gyb, acc)
        return acc

    acc = (zero, zero, zero)
    for s in range(3):
        acc = strip_pass(s, acc)

    accb[pl.ds(0, L)] = acc[0]
    accb[pl.ds(L, L)] = acc[1]
    accb[pl.ds(2 * L, L)] = acc[2]
    pltpu.sync_copy(accb, out_hbm.at[pl.ds(wid * 3 * L, 3 * L)])


@functools.partial(
    pl.kernel,
    out_type=jax.ShapeDtypeStruct((NW * 3 * L,), jnp.float32),
    mesh=plsc.VectorSubcoreMesh(core_axis_name="c", subcore_axis_name="s"),
    compiler_params=pltpu.CompilerParams(needs_layout_passes=False),
    scratch_types=[
        pltpu.VMEM((RING, CW), jnp.float32),
        pltpu.VMEM((RING, CW), jnp.float32),
        pltpu.VMEM((C, 8, SW), jnp.int32),
        pltpu.VMEM((C, 8, SW), jnp.int32),
        pltpu.VMEM((C, 8, SW), jnp.int32),
        pltpu.VMEM((C, 8, SW), jnp.int32),
        pltpu.VMEM((3 * L,), jnp.float32),
        pltpu.SemaphoreType.DMA,
        pltpu.SemaphoreType.DMA,
        pltpu.SemaphoreType.DMA,
    ],
)
def _depth_loss_partials(gs_hbm, gt_hbm, pr_hbm, out_hbm, gtb, prb,
                         gxa, gya, gxb, gyb, accb, sema, semb, semg):
    _sc_body(gs_hbm, gt_hbm, pr_hbm, out_hbm, gtb, prb, gxa, gya,
             gxb, gyb, accb, sema, semb, semg)


def kernel(pred_depth, gt_depth, grid, grid_shift):
    gs4 = jnp.transpose(grid_shift, (0, 3, 1, 2))
    parts = _depth_loss_partials(gs4, gt_depth, pred_depth)
    parts = parts.reshape(NW, 3, L)
    n_nz = jnp.sum(parts[:, 0])
    s_soft = jnp.sum(parts[:, 1])
    s_sq = jnp.sum(parts[:, 2])
    total = jnp.float32(H * W * C)
    depth_loss = s_soft / jnp.maximum(n_nz, 1.0)
    depth_loss_sim = s_sq / jnp.maximum(total - n_nz, 1.0)
    return depth_loss + depth_loss_sim


# speedup vs baseline: 65.3473x; 1.0342x over previous
import functools

import jax
import jax.numpy as jnp
from jax import lax
from jax.experimental import pallas as pl
from jax.experimental.pallas import tpu as pltpu
from jax.experimental.pallas import tpu_sc as plsc

H, W = 1080, 1920
C = 3
L = 16
RING = 32
SW = 640
CW = 896
CS = (0, 512, 1024)
SLOC = (0, 128, 256)
JG = SW // L
NC, NS = 2, 16
NW = NC * NS
NGRP = 6
TOL = 0.05

LOG2E = 1.4426950408889634
CE = (0.9999961199303905, 0.6930292690219008, 0.23938500062629817,
      0.05318647048254017, 0.006838262187515184)
CL = (2.2132784000594707e-05, 0.9990102089269637, -0.4891557820114497,
      0.28330238362040977, -0.1301179302884552, 0.030102247599666062)


def _softplus_terms(tg, tp, sg, sgx, sp):
    tgp = tg + 1e-8
    pos = sg >= (1.0 + TOL) * tgp
    neg = sgx <= tgp
    nz = pos | neg
    diff = sp - tp
    ad = jnp.abs(diff)
    x = jnp.maximum(ad * (-LOG2E), -126.0)
    ki = x.astype(jnp.int32)
    fr = x - ki.astype(jnp.float32)
    p2 = jnp.float32(CE[4])
    for c in (CE[3], CE[2], CE[1], CE[0]):
        p2 = p2 * fr + c
    scale = plsc.bitcast((ki + 127) << 23, jnp.float32)
    e = p2 * scale
    l = jnp.float32(CL[5])
    for c in (CL[4], CL[3], CL[2], CL[1], CL[0]):
        l = l * e + c
    t = jnp.where(pos, -diff, diff)
    soft = jnp.maximum(t, 0.0) + l
    m = jnp.where(nz, 1.0, 0.0)
    s = jnp.where(nz, soft, 0.0)
    q = jnp.where(nz, 0.0, diff * diff)
    return m, s, q


def _sc_body(gs_hbm, gt_hbm, pr_hbm, out_hbm, gtb, prb, gxa, gya,
             gxb, gyb, accb, sema, semb, semg):
    cid = lax.axis_index("c")
    sid = lax.axis_index("s")
    wid = sid * NC + cid

    lo = (wid * H) // NW
    hi = ((wid + 1) * H) // NW
    glo = lo // 8

    zero = jnp.zeros((L,), jnp.float32)

    def strip_pass(s, acc):
        cs = pl.multiple_of(s * 512, 128)
        sloc = s * 128
        scol = pl.multiple_of(s * SW, 128)

        def ring_dma(g):
            g = pl.multiple_of(g, 8)
            slot = pl.multiple_of(g & (RING - 1), 8)
            return [
                pltpu.make_async_copy(gt_hbm.at[pl.ds(g, 8), pl.ds(cs, CW)],
                                      gtb.at[pl.ds(slot, 8)], semg),
                pltpu.make_async_copy(pr_hbm.at[pl.ds(g, 8), pl.ds(cs, CW)],
                                      prb.at[pl.ds(slot, 8)], semg),
            ]

        def load_ring_group(g):
            for d in ring_dma(g):
                d.start()
            for d in ring_dma(g):
                d.wait()

        def idx_dma(gi, bufx, bufy, sem):
            g8 = pl.multiple_of(jnp.minimum(gi * 8, H - 8), 8)
            ds = []
            for c in range(C):
                ds.append(pltpu.make_async_copy(
                    gs_hbm.at[0, c, pl.ds(g8, 8), pl.ds(scol, SW)],
                    bufx.at[c], sem))
                ds.append(pltpu.make_async_copy(
                    gs_hbm.at[1, c, pl.ds(g8, 8), pl.ds(scol, SW)],
                    bufy.at[c], sem))
            return ds

        def idx_start(gi, bufx, bufy, sem):
            for d in idx_dma(gi, bufx, bufy, sem):
                d.start()

        def idx_wait(gi, bufx, bufy, sem):
            for d in idx_dma(gi, bufx, bufy, sem):
                d.wait()

        pg0 = jnp.maximum(lo - 10, 0) // 8
        pmax = (lo + 13) // 8

        def pre_body(gi, _):
            load_ring_group(gi * 8)
            return 0

        lax.fori_loop(pg0, pmax + 1, pre_body, 0)

        def process_group(gi, bufx, bufy, acc):
            gbase = gi * 8
            rlo = jnp.maximum(lo, gbase)
            rhi = jnp.minimum(hi, gbase + 8)

            def row_body(r, carry):
                @pl.when((((r + 14) & 7) == 0) & ((r + 14) // 8 > pmax)
                         & (r + 14 <= H - 8) & (r + 4 < hi))
                def _():
                    for d in ring_dma(r + 14):
                        d.start()

                @pl.when((((r + 10) & 7) == 0) & ((r + 10) // 8 > pmax)
                         & (r + 10 <= H - 8))
                def _():
                    for d in ring_dma(r + 10):
                        d.wait()

                rloc = r - gbase
                slot_r = r & (RING - 1)

                def grp(j, a):
                    a0, a1, a2 = a
                    jcol = j * L
                    sg = gtb[slot_r, pl.ds(sloc + jcol, L)]
                    sp = prb[slot_r, pl.ds(sloc + jcol, L)]
                    sgx = (1.0 + TOL) * sg
                    for c in range(C):
                        gxv = bufx[c, rloc, pl.ds(jcol, L)]
                        gyv = bufy[c, rloc, pl.ds(jcol, L)]
                        lslot = gyv & (RING - 1)
                        lcol = gxv - cs
                        tg = plsc.load_gather(gtb, [lslot, lcol])
                        tp = plsc.load_gather(prb, [lslot, lcol])
                        m, sf, q = _softplus_terms(tg, tp, sg, sgx, sp)
                        a0 = a0 + m
                        a1 = a1 + sf
                        a2 = a2 + q
                    return a0, a1, a2

                return plsc.parallel_loop(0, JG, unroll=4, carry=carry)(grp)

            return lax.fori_loop(rlo, rhi, row_body, acc)

        idx_start(glo, gxa, gya, sema)
        for gp in range(NGRP // 2):
            ga = glo + 2 * gp
            gb = ga + 1
            idx_start(gb, gxb, gyb, semb)
            idx_wait(ga, gxa, gya, sema)
            acc = process_group(ga, gxa, gya, acc)
            if gp < NGRP // 2 - 1:
                idx_start(ga + 2, gxa, gya, sema)
            idx_wait(gb, gxb, gyb, semb)
            acc = process_group(gb, gxb, gyb, acc)
        return acc

    acc = lax.fori_loop(0, 3, strip_pass, (zero, zero, zero))

    accb[pl.ds(0, L)] = acc[0]
    accb[pl.ds(L, L)] = acc[1]
    accb[pl.ds(2 * L, L)] = acc[2]
    pltpu.sync_copy(accb, out_hbm.at[pl.ds(wid * 3 * L, 3 * L)])


@functools.partial(
    pl.kernel,
    out_type=jax.ShapeDtypeStruct((NW * 3 * L,), jnp.float32),
    mesh=plsc.VectorSubcoreMesh(core_axis_name="c", subcore_axis_name="s"),
    compiler_params=pltpu.CompilerParams(needs_layout_passes=False),
    scratch_types=[
        pltpu.VMEM((RING, CW), jnp.float32),
        pltpu.VMEM((RING, CW), jnp.float32),
        pltpu.VMEM((C, 8, SW), jnp.int32),
        pltpu.VMEM((C, 8, SW), jnp.int32),
        pltpu.VMEM((C, 8, SW), jnp.int32),
        pltpu.VMEM((C, 8, SW), jnp.int32),
        pltpu.VMEM((3 * L,), jnp.float32),
        pltpu.SemaphoreType.DMA,
        pltpu.SemaphoreType.DMA,
        pltpu.SemaphoreType.DMA,
    ],
)
def _depth_loss_partials(gs_hbm, gt_hbm, pr_hbm, out_hbm, gtb, prb,
                         gxa, gya, gxb, gyb, accb, sema, semb, semg):
    _sc_body(gs_hbm, gt_hbm, pr_hbm, out_hbm, gtb, prb, gxa, gya,
             gxb, gyb, accb, sema, semb, semg)


def kernel(pred_depth, gt_depth, grid, grid_shift):
    gs4 = jnp.transpose(grid_shift, (0, 3, 1, 2))
    parts = _depth_loss_partials(gs4, gt_depth, pred_depth)
    parts = parts.reshape(NW, 3, L)
    n_nz = jnp.sum(parts[:, 0])
    s_soft = jnp.sum(parts[:, 1])
    s_sq = jnp.sum(parts[:, 2])
    total = jnp.float32(H * W * C)
    depth_loss = s_soft / jnp.maximum(n_nz, 1.0)
    depth_loss_sim = s_sq / jnp.maximum(total - n_nz, 1.0)
    return depth_loss + depth_loss_sim
